# Initial kernel scaffold; baseline (speedup 1.0000x reference)
import jax
import jax.numpy as jnp
from jax.experimental import pallas as pl


def kernel(x, edge_index, batch, W_enc1, b_enc1, W_enc2, b_enc2, W_g1, b_g1, W_g2, b_g2, W_g3, b_g3, W_sg, b_sg, W_fc1, b_fc1, W_fc2, b_fc2, W_cpd, b_cpd, W_comb, b_comb):
    raise NotImplementedError("write your pallas kernel here")


# speedup vs baseline: 10.6986x; 10.6986x over previous
import functools

import jax
import jax.numpy as jnp
from jax import lax
from jax.experimental import pallas as pl
from jax.experimental.pallas import tpu as pltpu
from jax.experimental.pallas import tpu_sc as plsc

NN = 10000
NPAD = 10240
EDG = 320000
NG = 64
NW = 32
CHUNK = 128
CPT = 79
EPAD = NW * CPT * CHUNK
SINK = NN
ROWS = NPAD // 16

_F32 = jnp.float32


def _sc_propagate(D):
    mesh = plsc.VectorSubcoreMesh(core_axis_name="c", subcore_axis_name="s")

    @functools.partial(
        pl.kernel,
        mesh=mesh,
        compiler_params=pltpu.CompilerParams(use_tc_tiling_on_sc=False),
        out_type=jax.ShapeDtypeStruct((2, NPAD, D), _F32),
        scratch_types=[
            pltpu.VMEM((CPT, CHUNK), jnp.int32),
            pltpu.VMEM((CPT, CHUNK), jnp.int32),
            pltpu.VMEM((CHUNK, D), _F32),
            pltpu.VMEM_SHARED((NPAD, D), _F32),
            pltpu.SemaphoreType.DMA,
        ],
    )
    def k(table, init0, init1, srcs, dsts, out, s_idx, d_idx, rows, acc, sem):
        c = lax.axis_index("c")
        s = lax.axis_index("s")
        wid = s * 2 + c
        rs = s * ROWS

        @pl.when(c == 0)
        def _():
            pltpu.sync_copy(init0.at[pl.ds(rs, ROWS)], acc.at[pl.ds(rs, ROWS)])

        @pl.when(c == 1)
        def _():
            pltpu.sync_copy(init1.at[pl.ds(rs, ROWS)], acc.at[pl.ds(rs, ROWS)])

        pltpu.sync_copy(srcs.at[wid], s_idx)
        pltpu.sync_copy(dsts.at[wid], d_idx)
        plsc.subcore_barrier()

        def body(j, carry):
            pltpu.async_copy(table.at[s_idx.at[j]], rows, sem).wait()
            pltpu.sync_copy(rows, acc.at[d_idx.at[j]], add=True)
            return carry

        lax.fori_loop(0, CPT, body, 0)
        plsc.subcore_barrier()
        pltpu.sync_copy(acc.at[pl.ds(rs, ROWS)], out.at[c, pl.ds(rs, ROWS)])

    return k


_R = 512


def _tc_deg_enc(degS, xp, W1):
    def body(degS_ref, x_ref, W_ref, dinv_ref, hs_ref):
        deg = degS_ref[0, :, 0:1] + degS_ref[1, :, 0:1]
        dinv = lax.rsqrt(jnp.maximum(deg, 1.0))
        dinv_ref[...] = dinv
        hs_ref[...] = dinv * jnp.dot(
            x_ref[...], W_ref[...], preferred_element_type=_F32)

    return pl.pallas_call(
        body,
        grid=(NPAD // _R,),
        in_specs=[
            pl.BlockSpec((2, _R, 16), lambda i: (0, i, 0)),
            pl.BlockSpec((_R, 128), lambda i: (i, 0)),
            pl.BlockSpec((128, 128), lambda i: (0, 0)),
        ],
        out_specs=[
            pl.BlockSpec((_R, 1), lambda i: (i, 0)),
            pl.BlockSpec((_R, 128), lambda i: (i, 0)),
        ],
        out_shape=[
            jax.ShapeDtypeStruct((NPAD, 1), _F32),
            jax.ShapeDtypeStruct((NPAD, 128), _F32),
        ],
    )(degS, xp, W1)


def _tc_step(S, dinv, pre_b, relu, W, post_b, scale_out):
    D = S.shape[2]
    D2 = W.shape[1] if W is not None else D
    operands = [S, dinv]
    in_specs = [
        pl.BlockSpec((2, _R, D), lambda i: (0, i, 0)),
        pl.BlockSpec((_R, 1), lambda i: (i, 0)),
    ]
    if pre_b is not None:
        operands.append(pre_b)
        in_specs.append(pl.BlockSpec((D,), lambda i: (0,)))
    if W is not None:
        operands.append(W)
        in_specs.append(pl.BlockSpec((D, D2), lambda i: (0, 0)))
    if post_b is not None:
        operands.append(post_b)
        in_specs.append(pl.BlockSpec((D2,), lambda i: (0,)))

    def body(S_ref, dinv_ref, *refs):
        refs = list(refs)
        o_ref = refs.pop()
        dinv = dinv_ref[...]
        t = dinv * (S_ref[0] + S_ref[1])
        if pre_b is not None:
            t = t + refs.pop(0)[...]
        if relu:
            t = jnp.maximum(t, 0.0)
        if W is not None:
            t = jnp.dot(t, refs.pop(0)[...], preferred_element_type=_F32)
        if post_b is not None:
            t = t + refs.pop(0)[...]
        if scale_out:
            t = dinv * t
        o_ref[...] = t

    return pl.pallas_call(
        body,
        grid=(NPAD // _R,),
        in_specs=in_specs,
        out_specs=pl.BlockSpec((_R, D2), lambda i: (i, 0)),
        out_shape=jax.ShapeDtypeStruct((NPAD, D2), _F32),
    )(*operands)


_RP = 256


def _tc_pool(t6, batchp):
    def body(t_ref, b_ref, o_ref):
        @pl.when(pl.program_id(0) == 0)
        def _():
            o_ref[...] = jnp.full((NG, 64), -jnp.inf, _F32)

        gids = lax.broadcasted_iota(jnp.int32, (_RP, NG, 1), 1)
        onehot = b_ref[...] == gids
        vals = jnp.where(onehot, t_ref[...], -jnp.inf)
        o_ref[...] = jnp.maximum(o_ref[...], jnp.max(vals, axis=0))

    return pl.pallas_call(
        body,
        grid=(NPAD // _RP,),
        in_specs=[
            pl.BlockSpec((_RP, 1, 64), lambda i: (i, 0, 0)),
            pl.BlockSpec((_RP, 1, 1), lambda i: (i, 0, 0)),
        ],
        out_specs=pl.BlockSpec((NG, 64), lambda i: (0, 0)),
        out_shape=jax.ShapeDtypeStruct((NG, 64), _F32),
    )(t6.reshape(NPAD, 1, 64), batchp.reshape(NPAD, 1, 1))


def _tc_mlp(p, W1, b1, W2, b2, Wc, bc, Wm, bm):
    def body(p_ref, W1r, b1r, W2r, b2r, Wcr, bcr, Wmr, bmr, o1, o2):
        z = jnp.maximum(jnp.dot(p_ref[...], W1r[...],
                                preferred_element_type=_F32) + b1r[...], 0.0)
        z = jnp.maximum(jnp.dot(z, W2r[...],
                                preferred_element_type=_F32) + b2r[...], 0.0)
        o1[...] = jnp.dot(z, Wcr[...], preferred_element_type=_F32) + bcr[...]
        o2[...] = jnp.dot(z, Wmr[...], preferred_element_type=_F32) + bmr[...]

    return pl.pallas_call(
        body,
        out_shape=[
            jax.ShapeDtypeStruct((NG, 1), _F32),
            jax.ShapeDtypeStruct((NG, 1), _F32),
        ],
    )(p, W1, b1, W2, b2, Wc, bc, Wm, bm)


def kernel(x, edge_index, batch, W_enc1, b_enc1, W_enc2, b_enc2, W_g1, b_g1,
           W_g2, b_g2, W_g3, b_g3, W_sg, b_sg, W_fc1, b_fc1, W_fc2, b_fc2,
           W_cpd, b_cpd, W_comb, b_comb):
    pad = EPAD - EDG
    srcs = jnp.concatenate(
        [edge_index[0], jnp.zeros((pad,), jnp.int32)]).reshape(NW, CPT, CHUNK)
    dsts = jnp.concatenate(
        [edge_index[1], jnp.full((pad,), SINK, jnp.int32)]).reshape(NW, CPT, CHUNK)

    ones16 = jnp.ones((NPAD, 16), _F32)
    zer16 = jnp.zeros((NPAD, 16), _F32)
    zer64 = jnp.zeros((NPAD, 64), _F32)
    zer128 = jnp.zeros((NPAD, 128), _F32)

    sc16 = _sc_propagate(16)
    sc64 = _sc_propagate(64)
    sc128 = _sc_propagate(128)

    degS = sc16(ones16, ones16, zer16, srcs, dsts)

    xp = jnp.pad(x, ((0, NPAD - NN), (0, 0)))
    dinv, hs = _tc_deg_enc(degS, xp, W_enc1)

    S = sc128(hs, hs, zer128, srcs, dsts)
    hs = _tc_step(S, dinv, b_enc1, True, W_enc2, None, True)
    S = sc64(hs, hs, zer64, srcs, dsts)
    hs = _tc_step(S, dinv, b_enc2, False, W_g1, None, True)
    S = sc64(hs, hs, zer64, srcs, dsts)
    hs = _tc_step(S, dinv, b_g1, True, W_g2, None, True)
    S = sc64(hs, hs, zer64, srcs, dsts)
    hs = _tc_step(S, dinv, b_g2, True, W_g3, None, True)
    S = sc64(hs, hs, zer64, srcs, dsts)
    hs = _tc_step(S, dinv, b_g3, True, None, None, True)
    S = sc64(hs, hs, zer64, srcs, dsts)
    hs = _tc_step(S, dinv, None, False, None, None, True)
    S = sc64(hs, hs, zer64, srcs, dsts)
    hs = _tc_step(S, dinv, None, False, None, None, True)
    S = sc64(hs, hs, zer64, srcs, dsts)
    hs = _tc_step(S, dinv, None, False, None, None, True)
    S = sc64(hs, hs, zer64, srcs, dsts)
    t6 = _tc_step(S, dinv, None, False, W_sg, b_sg, False)

    batchp = jnp.pad(batch, (0, NPAD - NN),
                     constant_values=NG).reshape(NPAD, 1)
    p = _tc_pool(t6, batchp)
    return _tc_mlp(p, W_fc1, b_fc1, W_fc2, b_fc2, W_cpd, b_cpd, W_comb, b_comb)
